# baseline (speedup 1.0000x reference)
import jax
import jax.numpy as jnp
from jax import lax
from jax.experimental import pallas as pl
from jax.experimental.pallas import tpu as pltpu

_TILE = 1024
_CHUNK = 64
_NSTREAM = 2
_STRIPE = _TILE // _NSTREAM


def _round_up(x: int, m: int) -> int:
    return (x + m - 1) // m * m


def _gmf_gather_kernel(ids_ref, u_tbl_ref, v_tbl_ref, out_hbm, buf, sems):
    i = pl.program_id(0)
    nsteps = pl.num_programs(0)
    slot = lax.rem(i, 2)

    def stripe_copy(step, slot_, s):
        row0 = step * _TILE + s * _STRIPE
        return pltpu.make_async_copy(
            buf.at[slot_, pl.ds(s * _STRIPE, _STRIPE), 0],
            out_hbm.at[pl.ds(row0, _STRIPE)],
            sems.at[slot_, s])

    @pl.when(i >= 2)
    def _wait_prev():
        for s in range(_NSTREAM):
            stripe_copy(i - 2, slot, s).wait()

    for g in range(_TILE // _CHUNK):
        base = g * _CHUNK
        u_rows = []
        v_rows = []
        for j in range(_CHUNK):
            w = ids_ref[0, 0, base + j]
            u_rows.append(u_tbl_ref[w & 4095, 0])
            v_rows.append(v_tbl_ref[w >> 12, 0])
        for j in range(_CHUNK):
            buf[slot, base + j, 0] = u_rows[j] * v_rows[j]

    for s in range(_NSTREAM):
        stripe_copy(i, slot, s).start(priority=s)

    @pl.when(i == nsteps - 1)
    def _drain_tail():
        @pl.when(nsteps >= 2)
        def _():
            for s in range(_NSTREAM):
                stripe_copy(i - 1, 1 - slot, s).wait()
        for s in range(_NSTREAM):
            stripe_copy(i, slot, s).wait()


@jax.jit
def kernel(u_idx, v_idx, u_table, v_table):
    batch = int(u_idx.shape[0])
    nu, emb = u_table.shape
    ni, emb_v = v_table.shape
    assert emb == emb_v, "embedding dims must match"
    out_dtype = jnp.result_type(u_table.dtype, v_table.dtype)

    u_idx = jnp.clip(u_idx.astype(jnp.int32), 0, nu - 1)
    v_idx = jnp.clip(v_idx.astype(jnp.int32), 0, ni - 1)
    packed = u_idx | (v_idx << 12)

    batch_pad = _round_up(batch, _TILE)
    if batch_pad != batch:
        packed = jnp.pad(packed, (0, batch_pad - batch))
    n_tiles = batch_pad // _TILE

    ids = packed.reshape(n_tiles, 1, _TILE)
    u_t3 = u_table.reshape(nu, 1, emb)
    v_t3 = v_table.reshape(ni, 1, emb)

    out = pl.pallas_call(
        _gmf_gather_kernel,
        out_shape=jax.ShapeDtypeStruct((batch_pad, emb), out_dtype),
        grid=(n_tiles,),
        in_specs=[
            pl.BlockSpec((1, 1, _TILE), lambda i: (i, 0, 0),
                         memory_space=pltpu.SMEM),
            pl.BlockSpec((nu, 1, emb), lambda i: (0, 0, 0)),
            pl.BlockSpec((ni, 1, emb), lambda i: (0, 0, 0)),
        ],
        out_specs=pl.BlockSpec(memory_space=pl.ANY),
        scratch_shapes=[
            pltpu.VMEM((2, _TILE, 1, emb), out_dtype),
            pltpu.SemaphoreType.DMA((2, _NSTREAM)),
        ],
        compiler_params=pltpu.CompilerParams(
            dimension_semantics=("arbitrary",),
            vmem_limit_bytes=56 * 1024 * 1024,
        ),
    )(ids, u_t3, v_t3)

    return out[:batch]


# speedup vs baseline: 1.0075x; 1.0075x over previous
import jax
import jax.numpy as jnp
from jax import lax
from jax.experimental import pallas as pl
from jax.experimental.pallas import tpu as pltpu

_TILE = 8192
_CHUNK = 64
_NSTREAM = 2
_STRIPE = _TILE // _NSTREAM


def _round_up(x: int, m: int) -> int:
    return (x + m - 1) // m * m


def _gmf_gather_kernel(ids_ref, u_tbl_ref, v_tbl_ref, out_hbm, buf, sems):
    i = pl.program_id(0)
    nsteps = pl.num_programs(0)
    slot = lax.rem(i, 2)

    def stripe_copy(step, slot_, s):
        row0 = step * _TILE + s * _STRIPE
        return pltpu.make_async_copy(
            buf.at[slot_, pl.ds(s * _STRIPE, _STRIPE), 0],
            out_hbm.at[pl.ds(row0, _STRIPE)],
            sems.at[slot_, s])

    @pl.when(i >= 2)
    def _wait_prev():
        for s in range(_NSTREAM):
            stripe_copy(i - 2, slot, s).wait()

    def chunk_body(c, carry):
        base = c * _CHUNK
        u_rows = []
        v_rows = []
        for j in range(_CHUNK):
            w = ids_ref[0, 0, base + j]
            u_rows.append(u_tbl_ref[w & 4095, 0])
            v_rows.append(v_tbl_ref[w >> 12, 0])
        for j in range(_CHUNK):
            buf[slot, pl.ds(base + j, 1), 0] = (u_rows[j] * v_rows[j])[None]
        return carry

    lax.fori_loop(0, _TILE // _CHUNK, chunk_body, 0)

    for s in range(_NSTREAM):
        stripe_copy(i, slot, s).start(priority=s)

    @pl.when(i == nsteps - 1)
    def _drain_tail():
        @pl.when(nsteps >= 2)
        def _():
            for s in range(_NSTREAM):
                stripe_copy(i - 1, 1 - slot, s).wait()
        for s in range(_NSTREAM):
            stripe_copy(i, slot, s).wait()


@jax.jit
def kernel(u_idx, v_idx, u_table, v_table):
    batch = int(u_idx.shape[0])
    nu, emb = u_table.shape
    ni, emb_v = v_table.shape
    assert emb == emb_v, "embedding dims must match"
    out_dtype = jnp.result_type(u_table.dtype, v_table.dtype)

    u_idx = jnp.clip(u_idx.astype(jnp.int32), 0, nu - 1)
    v_idx = jnp.clip(v_idx.astype(jnp.int32), 0, ni - 1)
    packed = u_idx | (v_idx << 12)

    batch_pad = _round_up(batch, _TILE)
    if batch_pad != batch:
        packed = jnp.pad(packed, (0, batch_pad - batch))
    n_tiles = batch_pad // _TILE

    ids = packed.reshape(n_tiles, 1, _TILE)
    u_t3 = u_table.reshape(nu, 1, emb)
    v_t3 = v_table.reshape(ni, 1, emb)

    out = pl.pallas_call(
        _gmf_gather_kernel,
        out_shape=jax.ShapeDtypeStruct((batch_pad, emb), out_dtype),
        grid=(n_tiles,),
        in_specs=[
            pl.BlockSpec((1, 1, _TILE), lambda i: (i, 0, 0),
                         memory_space=pltpu.SMEM),
            pl.BlockSpec((nu, 1, emb), lambda i: (0, 0, 0)),
            pl.BlockSpec((ni, 1, emb), lambda i: (0, 0, 0)),
        ],
        out_specs=pl.BlockSpec(memory_space=pl.ANY),
        scratch_shapes=[
            pltpu.VMEM((2, _TILE, 1, emb), out_dtype),
            pltpu.SemaphoreType.DMA((2, _NSTREAM)),
        ],
        compiler_params=pltpu.CompilerParams(
            dimension_semantics=("arbitrary",),
            vmem_limit_bytes=56 * 1024 * 1024,
        ),
    )(ids, u_t3, v_t3)

    return out[:batch]


# speedup vs baseline: 1.0078x; 1.0003x over previous
import jax
import jax.numpy as jnp
from jax import lax
from jax.experimental import pallas as pl
from jax.experimental.pallas import tpu as pltpu

_TILE = 4096
_CHUNK = 64
_NSTREAM = 2
_STRIPE = _TILE // _NSTREAM


def _round_up(x: int, m: int) -> int:
    return (x + m - 1) // m * m


def _make_gmf_kernel(nbits: int):
    mask = (1 << nbits) - 1

    def _gmf_gather_kernel(ids_ref, u_tbl_ref, v_tbl_ref, out_hbm, buf, sems):
        i = pl.program_id(0)
        nsteps = pl.num_programs(0)
        slot = lax.rem(i, 2)

        def stripe_copy(step, slot_, s):
            row0 = step * _TILE + s * _STRIPE
            return pltpu.make_async_copy(
                buf.at[slot_, pl.ds(s * _STRIPE, _STRIPE), 0],
                out_hbm.at[pl.ds(row0, _STRIPE)],
                sems.at[slot_, s])

        @pl.when(i >= 2)
        def _wait_prev():
            for s in range(_NSTREAM):
                stripe_copy(i - 2, slot, s).wait()

        def chunk_body(c, carry):
            base = c * _CHUNK
            u_rows = []
            v_rows = []
            for j in range(_CHUNK):
                w = ids_ref[0, 0, base + j]
                u_rows.append(u_tbl_ref[w & mask, 0])
                v_rows.append(v_tbl_ref[w >> nbits, 0])
            for j in range(_CHUNK):
                buf[slot, pl.ds(base + j, 1), 0] = (u_rows[j] * v_rows[j])[None]
            return carry

        lax.fori_loop(0, _TILE // _CHUNK, chunk_body, 0)

        for s in range(_NSTREAM):
            stripe_copy(i, slot, s).start(priority=s)

        @pl.when(i == nsteps - 1)
        def _drain_tail():
            @pl.when(nsteps >= 2)
            def _():
                for s in range(_NSTREAM):
                    stripe_copy(i - 1, 1 - slot, s).wait()
            for s in range(_NSTREAM):
                stripe_copy(i, slot, s).wait()

    return _gmf_gather_kernel


@jax.jit
def kernel(u_idx, v_idx, u_table, v_table):
    batch = int(u_idx.shape[0])
    nu, emb = u_table.shape
    ni, emb_v = v_table.shape
    assert emb == emb_v, "embedding dims must match"
    out_dtype = jnp.result_type(u_table.dtype, v_table.dtype)

    nbits = max(1, int(nu - 1).bit_length())
    assert nbits + max(1, int(ni - 1).bit_length()) <= 31, "indices too wide"
    u_idx = jnp.clip(u_idx.astype(jnp.int32), 0, nu - 1)
    v_idx = jnp.clip(v_idx.astype(jnp.int32), 0, ni - 1)
    packed = u_idx | (v_idx << nbits)

    batch_pad = _round_up(batch, _TILE)
    if batch_pad != batch:
        packed = jnp.pad(packed, (0, batch_pad - batch))
    n_tiles = batch_pad // _TILE

    ids = packed.reshape(n_tiles, 1, _TILE)
    u_t3 = u_table.reshape(nu, 1, emb)
    v_t3 = v_table.reshape(ni, 1, emb)

    out = pl.pallas_call(
        _make_gmf_kernel(nbits),
        out_shape=jax.ShapeDtypeStruct((batch_pad, emb), out_dtype),
        grid=(n_tiles,),
        in_specs=[
            pl.BlockSpec((1, 1, _TILE), lambda i: (i, 0, 0),
                         memory_space=pltpu.SMEM),
            pl.BlockSpec((nu, 1, emb), lambda i: (0, 0, 0)),
            pl.BlockSpec((ni, 1, emb), lambda i: (0, 0, 0)),
        ],
        out_specs=pl.BlockSpec(memory_space=pl.ANY),
        scratch_shapes=[
            pltpu.VMEM((2, _TILE, 1, emb), out_dtype),
            pltpu.SemaphoreType.DMA((2, _NSTREAM)),
        ],
        compiler_params=pltpu.CompilerParams(
            dimension_semantics=("arbitrary",),
            vmem_limit_bytes=56 * 1024 * 1024,
        ),
    )(ids, u_t3, v_t3)

    return out[:batch]


# speedup vs baseline: 1.0269x; 1.0189x over previous
import jax
import jax.numpy as jnp
from jax import lax
from jax.experimental import pallas as pl
from jax.experimental.pallas import tpu as pltpu

_TILE = 4096
_CHUNK = 128
_NSTREAM = 2
_STRIPE = _TILE // _NSTREAM


def _round_up(x: int, m: int) -> int:
    return (x + m - 1) // m * m


def _make_gmf_kernel(nbits: int):
    mask = (1 << nbits) - 1

    def _gmf_gather_kernel(ids_ref, u_tbl_ref, v_tbl_ref, out_hbm, buf, sems):
        i = pl.program_id(0)
        nsteps = pl.num_programs(0)
        slot = lax.rem(i, 2)

        def stripe_copy(step, slot_, s):
            row0 = step * _TILE + s * _STRIPE
            return pltpu.make_async_copy(
                buf.at[slot_, pl.ds(s * _STRIPE, _STRIPE), 0],
                out_hbm.at[pl.ds(row0, _STRIPE)],
                sems.at[slot_, s])

        @pl.when(i >= 2)
        def _wait_prev():
            for s in range(_NSTREAM):
                stripe_copy(i - 2, slot, s).wait()

        def chunk_body(c, carry):
            base = c * _CHUNK
            u_rows = []
            v_rows = []
            for j in range(_CHUNK):
                w = ids_ref[0, 0, base + j]
                u_rows.append(u_tbl_ref[w & mask, 0])
                v_rows.append(v_tbl_ref[w >> nbits, 0])
            for j in range(_CHUNK):
                buf[slot, pl.ds(base + j, 1), 0] = (u_rows[j] * v_rows[j])[None]
            return carry

        lax.fori_loop(0, _TILE // _CHUNK, chunk_body, 0)

        for s in range(_NSTREAM):
            stripe_copy(i, slot, s).start(priority=s)

        @pl.when(i == nsteps - 1)
        def _drain_tail():
            @pl.when(nsteps >= 2)
            def _():
                for s in range(_NSTREAM):
                    stripe_copy(i - 1, 1 - slot, s).wait()
            for s in range(_NSTREAM):
                stripe_copy(i, slot, s).wait()

    return _gmf_gather_kernel


@jax.jit
def kernel(u_idx, v_idx, u_table, v_table):
    batch = int(u_idx.shape[0])
    nu, emb = u_table.shape
    ni, emb_v = v_table.shape
    assert emb == emb_v, "embedding dims must match"
    out_dtype = jnp.result_type(u_table.dtype, v_table.dtype)

    nbits = max(1, int(nu - 1).bit_length())
    assert nbits + max(1, int(ni - 1).bit_length()) <= 31, "indices too wide"
    u_idx = jnp.clip(u_idx.astype(jnp.int32), 0, nu - 1)
    v_idx = jnp.clip(v_idx.astype(jnp.int32), 0, ni - 1)
    packed = u_idx | (v_idx << nbits)

    batch_pad = _round_up(batch, _TILE)
    if batch_pad != batch:
        packed = jnp.pad(packed, (0, batch_pad - batch))
    n_tiles = batch_pad // _TILE

    ids = packed.reshape(n_tiles, 1, _TILE)
    u_t3 = u_table.reshape(nu, 1, emb)
    v_t3 = v_table.reshape(ni, 1, emb)

    out = pl.pallas_call(
        _make_gmf_kernel(nbits),
        out_shape=jax.ShapeDtypeStruct((batch_pad, emb), out_dtype),
        grid=(n_tiles,),
        in_specs=[
            pl.BlockSpec((1, 1, _TILE), lambda i: (i, 0, 0),
                         memory_space=pltpu.SMEM),
            pl.BlockSpec((nu, 1, emb), lambda i: (0, 0, 0)),
            pl.BlockSpec((ni, 1, emb), lambda i: (0, 0, 0)),
        ],
        out_specs=pl.BlockSpec(memory_space=pl.ANY),
        scratch_shapes=[
            pltpu.VMEM((2, _TILE, 1, emb), out_dtype),
            pltpu.SemaphoreType.DMA((2, _NSTREAM)),
        ],
        compiler_params=pltpu.CompilerParams(
            dimension_semantics=("arbitrary",),
            vmem_limit_bytes=56 * 1024 * 1024,
        ),
    )(ids, u_t3, v_t3)

    return out[:batch]


# speedup vs baseline: 1.0428x; 1.0155x over previous
import jax
import jax.numpy as jnp
from jax import lax
from jax.experimental import pallas as pl
from jax.experimental.pallas import tpu as pltpu

_TILE = 4096
_CHUNK = 256
_NSTREAM = 2
_STRIPE = _TILE // _NSTREAM


def _round_up(x: int, m: int) -> int:
    return (x + m - 1) // m * m


def _make_gmf_kernel(nbits: int):
    mask = (1 << nbits) - 1

    def _gmf_gather_kernel(ids_ref, u_tbl_ref, v_tbl_ref, out_hbm, buf, sems):
        i = pl.program_id(0)
        nsteps = pl.num_programs(0)
        slot = lax.rem(i, 2)

        def stripe_copy(step, slot_, s):
            row0 = step * _TILE + s * _STRIPE
            return pltpu.make_async_copy(
                buf.at[slot_, pl.ds(s * _STRIPE, _STRIPE), 0],
                out_hbm.at[pl.ds(row0, _STRIPE)],
                sems.at[slot_, s])

        @pl.when(i >= 2)
        def _wait_prev():
            for s in range(_NSTREAM):
                stripe_copy(i - 2, slot, s).wait()

        def chunk_body(c, carry):
            base = c * _CHUNK
            u_rows = []
            v_rows = []
            for j in range(_CHUNK):
                w = ids_ref[0, 0, base + j]
                u_rows.append(u_tbl_ref[w & mask, 0])
                v_rows.append(v_tbl_ref[w >> nbits, 0])
            for j in range(_CHUNK):
                buf[slot, pl.ds(base + j, 1), 0] = (u_rows[j] * v_rows[j])[None]
            return carry

        lax.fori_loop(0, _TILE // _CHUNK, chunk_body, 0)

        for s in range(_NSTREAM):
            stripe_copy(i, slot, s).start(priority=s)

        @pl.when(i == nsteps - 1)
        def _drain_tail():
            @pl.when(nsteps >= 2)
            def _():
                for s in range(_NSTREAM):
                    stripe_copy(i - 1, 1 - slot, s).wait()
            for s in range(_NSTREAM):
                stripe_copy(i, slot, s).wait()

    return _gmf_gather_kernel


@jax.jit
def kernel(u_idx, v_idx, u_table, v_table):
    batch = int(u_idx.shape[0])
    nu, emb = u_table.shape
    ni, emb_v = v_table.shape
    assert emb == emb_v, "embedding dims must match"
    out_dtype = jnp.result_type(u_table.dtype, v_table.dtype)

    nbits = max(1, int(nu - 1).bit_length())
    assert nbits + max(1, int(ni - 1).bit_length()) <= 31, "indices too wide"
    u_idx = jnp.clip(u_idx.astype(jnp.int32), 0, nu - 1)
    v_idx = jnp.clip(v_idx.astype(jnp.int32), 0, ni - 1)
    packed = u_idx | (v_idx << nbits)

    batch_pad = _round_up(batch, _TILE)
    if batch_pad != batch:
        packed = jnp.pad(packed, (0, batch_pad - batch))
    n_tiles = batch_pad // _TILE

    ids = packed.reshape(n_tiles, 1, _TILE)
    u_t3 = u_table.reshape(nu, 1, emb)
    v_t3 = v_table.reshape(ni, 1, emb)

    out = pl.pallas_call(
        _make_gmf_kernel(nbits),
        out_shape=jax.ShapeDtypeStruct((batch_pad, emb), out_dtype),
        grid=(n_tiles,),
        in_specs=[
            pl.BlockSpec((1, 1, _TILE), lambda i: (i, 0, 0),
                         memory_space=pltpu.SMEM),
            pl.BlockSpec((nu, 1, emb), lambda i: (0, 0, 0)),
            pl.BlockSpec((ni, 1, emb), lambda i: (0, 0, 0)),
        ],
        out_specs=pl.BlockSpec(memory_space=pl.ANY),
        scratch_shapes=[
            pltpu.VMEM((2, _TILE, 1, emb), out_dtype),
            pltpu.SemaphoreType.DMA((2, _NSTREAM)),
        ],
        compiler_params=pltpu.CompilerParams(
            dimension_semantics=("arbitrary",),
            vmem_limit_bytes=56 * 1024 * 1024,
        ),
    )(ids, u_t3, v_t3)

    return out[:batch]


# speedup vs baseline: 1.0467x; 1.0038x over previous
import jax
import jax.numpy as jnp
from jax import lax
from jax.experimental import pallas as pl
from jax.experimental.pallas import tpu as pltpu

_TILE = 4096
_CHUNK = 512
_NSTREAM = 2
_STRIPE = _TILE // _NSTREAM


def _round_up(x: int, m: int) -> int:
    return (x + m - 1) // m * m


def _make_gmf_kernel(nbits: int):
    mask = (1 << nbits) - 1

    def _gmf_gather_kernel(ids_ref, u_tbl_ref, v_tbl_ref, out_hbm, buf, sems):
        i = pl.program_id(0)
        nsteps = pl.num_programs(0)
        slot = lax.rem(i, 2)

        def stripe_copy(step, slot_, s):
            row0 = step * _TILE + s * _STRIPE
            return pltpu.make_async_copy(
                buf.at[slot_, pl.ds(s * _STRIPE, _STRIPE), 0],
                out_hbm.at[pl.ds(row0, _STRIPE)],
                sems.at[slot_, s])

        @pl.when(i >= 2)
        def _wait_prev():
            for s in range(_NSTREAM):
                stripe_copy(i - 2, slot, s).wait()

        def chunk_body(c, carry):
            base = c * _CHUNK
            u_rows = []
            v_rows = []
            for j in range(_CHUNK):
                w = ids_ref[0, 0, base + j]
                u_rows.append(u_tbl_ref[w & mask, 0])
                v_rows.append(v_tbl_ref[w >> nbits, 0])
            for j in range(_CHUNK):
                buf[slot, pl.ds(base + j, 1), 0] = (u_rows[j] * v_rows[j])[None]
            return carry

        lax.fori_loop(0, _TILE // _CHUNK, chunk_body, 0)

        for s in range(_NSTREAM):
            stripe_copy(i, slot, s).start(priority=s)

        @pl.when(i == nsteps - 1)
        def _drain_tail():
            @pl.when(nsteps >= 2)
            def _():
                for s in range(_NSTREAM):
                    stripe_copy(i - 1, 1 - slot, s).wait()
            for s in range(_NSTREAM):
                stripe_copy(i, slot, s).wait()

    return _gmf_gather_kernel


@jax.jit
def kernel(u_idx, v_idx, u_table, v_table):
    batch = int(u_idx.shape[0])
    nu, emb = u_table.shape
    ni, emb_v = v_table.shape
    assert emb == emb_v, "embedding dims must match"
    out_dtype = jnp.result_type(u_table.dtype, v_table.dtype)

    nbits = max(1, int(nu - 1).bit_length())
    assert nbits + max(1, int(ni - 1).bit_length()) <= 31, "indices too wide"
    u_idx = jnp.clip(u_idx.astype(jnp.int32), 0, nu - 1)
    v_idx = jnp.clip(v_idx.astype(jnp.int32), 0, ni - 1)
    packed = u_idx | (v_idx << nbits)

    batch_pad = _round_up(batch, _TILE)
    if batch_pad != batch:
        packed = jnp.pad(packed, (0, batch_pad - batch))
    n_tiles = batch_pad // _TILE

    ids = packed.reshape(n_tiles, 1, _TILE)
    u_t3 = u_table.reshape(nu, 1, emb)
    v_t3 = v_table.reshape(ni, 1, emb)

    out = pl.pallas_call(
        _make_gmf_kernel(nbits),
        out_shape=jax.ShapeDtypeStruct((batch_pad, emb), out_dtype),
        grid=(n_tiles,),
        in_specs=[
            pl.BlockSpec((1, 1, _TILE), lambda i: (i, 0, 0),
                         memory_space=pltpu.SMEM),
            pl.BlockSpec((nu, 1, emb), lambda i: (0, 0, 0)),
            pl.BlockSpec((ni, 1, emb), lambda i: (0, 0, 0)),
        ],
        out_specs=pl.BlockSpec(memory_space=pl.ANY),
        scratch_shapes=[
            pltpu.VMEM((2, _TILE, 1, emb), out_dtype),
            pltpu.SemaphoreType.DMA((2, _NSTREAM)),
        ],
        compiler_params=pltpu.CompilerParams(
            dimension_semantics=("arbitrary",),
            vmem_limit_bytes=56 * 1024 * 1024,
        ),
    )(ids, u_t3, v_t3)

    return out[:batch]


# speedup vs baseline: 1.0514x; 1.0045x over previous
import jax
import jax.numpy as jnp
from jax import lax
from jax.experimental import pallas as pl
from jax.experimental.pallas import tpu as pltpu

_TILE = 4096
_CHUNK = 1024
_NSTREAM = 2
_STRIPE = _TILE // _NSTREAM


def _round_up(x: int, m: int) -> int:
    return (x + m - 1) // m * m


def _make_gmf_kernel(nbits: int):
    mask = (1 << nbits) - 1

    def _gmf_gather_kernel(ids_ref, u_tbl_ref, v_tbl_ref, out_hbm, buf, sems):
        i = pl.program_id(0)
        nsteps = pl.num_programs(0)
        slot = lax.rem(i, 2)

        def stripe_copy(step, slot_, s):
            row0 = step * _TILE + s * _STRIPE
            return pltpu.make_async_copy(
                buf.at[slot_, pl.ds(s * _STRIPE, _STRIPE), 0],
                out_hbm.at[pl.ds(row0, _STRIPE)],
                sems.at[slot_, s])

        @pl.when(i >= 2)
        def _wait_prev():
            for s in range(_NSTREAM):
                stripe_copy(i - 2, slot, s).wait()

        def chunk_body(c, carry):
            base = c * _CHUNK
            u_rows = []
            v_rows = []
            for j in range(_CHUNK):
                w = ids_ref[0, 0, base + j]
                u_rows.append(u_tbl_ref[w & mask, 0])
                v_rows.append(v_tbl_ref[w >> nbits, 0])
            for j in range(_CHUNK):
                buf[slot, pl.ds(base + j, 1), 0] = (u_rows[j] * v_rows[j])[None]
            return carry

        lax.fori_loop(0, _TILE // _CHUNK, chunk_body, 0)

        for s in range(_NSTREAM):
            stripe_copy(i, slot, s).start(priority=s)

        @pl.when(i == nsteps - 1)
        def _drain_tail():
            @pl.when(nsteps >= 2)
            def _():
                for s in range(_NSTREAM):
                    stripe_copy(i - 1, 1 - slot, s).wait()
            for s in range(_NSTREAM):
                stripe_copy(i, slot, s).wait()

    return _gmf_gather_kernel


@jax.jit
def kernel(u_idx, v_idx, u_table, v_table):
    batch = int(u_idx.shape[0])
    nu, emb = u_table.shape
    ni, emb_v = v_table.shape
    assert emb == emb_v, "embedding dims must match"
    out_dtype = jnp.result_type(u_table.dtype, v_table.dtype)

    nbits = max(1, int(nu - 1).bit_length())
    assert nbits + max(1, int(ni - 1).bit_length()) <= 31, "indices too wide"
    u_idx = jnp.clip(u_idx.astype(jnp.int32), 0, nu - 1)
    v_idx = jnp.clip(v_idx.astype(jnp.int32), 0, ni - 1)
    packed = u_idx | (v_idx << nbits)

    batch_pad = _round_up(batch, _TILE)
    if batch_pad != batch:
        packed = jnp.pad(packed, (0, batch_pad - batch))
    n_tiles = batch_pad // _TILE

    ids = packed.reshape(n_tiles, 1, _TILE)
    u_t3 = u_table.reshape(nu, 1, emb)
    v_t3 = v_table.reshape(ni, 1, emb)

    out = pl.pallas_call(
        _make_gmf_kernel(nbits),
        out_shape=jax.ShapeDtypeStruct((batch_pad, emb), out_dtype),
        grid=(n_tiles,),
        in_specs=[
            pl.BlockSpec((1, 1, _TILE), lambda i: (i, 0, 0),
                         memory_space=pltpu.SMEM),
            pl.BlockSpec((nu, 1, emb), lambda i: (0, 0, 0)),
            pl.BlockSpec((ni, 1, emb), lambda i: (0, 0, 0)),
        ],
        out_specs=pl.BlockSpec(memory_space=pl.ANY),
        scratch_shapes=[
            pltpu.VMEM((2, _TILE, 1, emb), out_dtype),
            pltpu.SemaphoreType.DMA((2, _NSTREAM)),
        ],
        compiler_params=pltpu.CompilerParams(
            dimension_semantics=("arbitrary",),
            vmem_limit_bytes=56 * 1024 * 1024,
        ),
    )(ids, u_t3, v_t3)

    return out[:batch]
